# Initial kernel scaffold; baseline (speedup 1.0000x reference)
import functools

import jax
import jax.numpy as jnp
from jax import lax
from jax.experimental import pallas as pl
from jax.experimental.pallas import tpu as pltpu
from jax.experimental.pallas import tpu_sc as plsc

N = 10000
E = 160000
F = 256
HF = 128
NCLS = 64
NC = 2
NS = 16
ROWS_PER_TILE = N // NS
K = 80
EDGES_PER_TILE = E // NS
DEG_EDGES_PER_TILE = E // (NC * NS)
KD = 40
RB = 2000

_mesh = plsc.VectorSubcoreMesh(core_axis_name="c", subcore_axis_name="s")


@functools.partial(
    pl.kernel,
    out_type=jax.ShapeDtypeStruct((NC, N, 16), jnp.float32),
    mesh=_mesh,
    scratch_types=[
        pltpu.VMEM_SHARED((N, 16), jnp.float32),
        pltpu.VMEM((ROWS_PER_TILE, 16), jnp.float32),
        pltpu.VMEM((KD, 16), jnp.float32),
        pltpu.VMEM((KD,), jnp.int32),
    ],
)
def _deg_kernel(dst_hbm, out_hbm, cnt_sh, zbuf, ones_v, didx):
    c = lax.axis_index("c")
    s = lax.axis_index("s")

    def fill_zero(i, carry):
        zbuf[i, :] = jnp.zeros((16,), jnp.float32)
        return carry

    lax.fori_loop(0, ROWS_PER_TILE, fill_zero, 0)

    def fill_one(i, carry):
        ones_v[i, :] = jnp.ones((16,), jnp.float32)
        return carry

    lax.fori_loop(0, KD, fill_one, 0)

    row0 = s * ROWS_PER_TILE
    pltpu.sync_copy(zbuf, cnt_sh.at[pl.ds(row0, ROWS_PER_TILE)])
    plsc.subcore_barrier()

    base = (c * NS + s) * DEG_EDGES_PER_TILE

    def body(i, carry):
        pltpu.sync_copy(dst_hbm.at[pl.ds(base + i * KD, KD)], didx)
        pltpu.sync_copy(ones_v, cnt_sh.at[didx], add=True)
        return carry

    lax.fori_loop(0, DEG_EDGES_PER_TILE // KD, body, 0)
    plsc.subcore_barrier()

    pltpu.sync_copy(cnt_sh.at[pl.ds(row0, ROWS_PER_TILE)],
                    out_hbm.at[c, pl.ds(row0, ROWS_PER_TILE)])


@functools.partial(
    pl.kernel,
    out_type=jax.ShapeDtypeStruct((2 * N, HF), jnp.float32),
    mesh=_mesh,
    scratch_types=[
        pltpu.VMEM_SHARED((N, HF), jnp.float32),
        pltpu.VMEM((K,), jnp.int32),
        pltpu.VMEM((K,), jnp.int32),
        pltpu.VMEM((K,), jnp.int32),
        pltpu.VMEM((K, HF), jnp.float32),
        pltpu.SemaphoreType.DMA,
    ],
)
def _scatter_kernel(hs_hbm, src_hbm, dst_hbm, out_hbm,
                    acc_sh, sidx, gidx, didx, gbuf, sem):
    c = lax.axis_index("c")
    s = lax.axis_index("s")
    row0 = s * ROWS_PER_TILE
    half = c * N

    pltpu.sync_copy(hs_hbm.at[pl.ds(half + row0, ROWS_PER_TILE)],
                    acc_sh.at[pl.ds(row0, ROWS_PER_TILE)])
    plsc.subcore_barrier()

    ebase = s * EDGES_PER_TILE

    def body(i, carry):
        b = ebase + i * K
        pltpu.sync_copy(src_hbm.at[pl.ds(b, K)], sidx)
        pltpu.sync_copy(dst_hbm.at[pl.ds(b, K)], didx)
        for j in range(K // 16):
            gidx[pl.ds(j * 16, 16)] = sidx[pl.ds(j * 16, 16)] + half
        pltpu.async_copy(hs_hbm.at[gidx], gbuf, sem).wait()
        pltpu.sync_copy(gbuf, acc_sh.at[didx], add=True)
        return carry

    lax.fori_loop(0, EDGES_PER_TILE // K, body, 0)
    plsc.subcore_barrier()

    pltpu.sync_copy(acc_sh.at[pl.ds(row0, ROWS_PER_TILE)],
                    out_hbm.at[pl.ds(half + row0, ROWS_PER_TILE)])


def _dinv_body(dw_ref, o_ref):
    deg = dw_ref[0][:, 0:1] + dw_ref[1][:, 0:1] + 1.0
    o_ref[...] = lax.rsqrt(deg)


_dinv_call = pl.pallas_call(
    _dinv_body,
    in_specs=[pl.BlockSpec((NC, N, 16), lambda: (0, 0, 0))],
    out_specs=pl.BlockSpec((N, 1), lambda: (0, 0)),
    out_shape=jax.ShapeDtypeStruct((N, 1), jnp.float32),
)


def _mm1_body(x_ref, w_ref, dinv_ref, o_ref):
    y = jnp.dot(x_ref[...], w_ref[...], preferred_element_type=jnp.float32)
    y = y * dinv_ref[...]
    o_ref[0] = y[:, :HF]
    o_ref[1] = y[:, HF:]


_mm1_call = pl.pallas_call(
    _mm1_body,
    grid=(N // RB,),
    in_specs=[
        pl.BlockSpec((RB, F), lambda i: (i, 0)),
        pl.BlockSpec((F, F), lambda i: (0, 0)),
        pl.BlockSpec((RB, 1), lambda i: (i, 0)),
    ],
    out_specs=pl.BlockSpec((NC, RB, HF), lambda i: (0, i, 0)),
    out_shape=jax.ShapeDtypeStruct((NC, N, HF), jnp.float32),
)


def _mid_body(agg_ref, dinv_ref, b_ref, w_ref, o_ref):
    dinv = dinv_ref[...]
    h = jnp.concatenate([agg_ref[0], agg_ref[1]], axis=1) * dinv + b_ref[...]
    y = jnp.dot(h, w_ref[...], preferred_element_type=jnp.float32) * dinv
    o_ref[0] = y[:, :HF]
    o_ref[1] = y[:, HF:]


_mid_call = pl.pallas_call(
    _mid_body,
    grid=(N // RB,),
    in_specs=[
        pl.BlockSpec((NC, RB, HF), lambda i: (0, i, 0)),
        pl.BlockSpec((RB, 1), lambda i: (i, 0)),
        pl.BlockSpec((1, F), lambda i: (0, 0)),
        pl.BlockSpec((F, F), lambda i: (0, 0)),
    ],
    out_specs=pl.BlockSpec((NC, RB, HF), lambda i: (0, i, 0)),
    out_shape=jax.ShapeDtypeStruct((NC, N, HF), jnp.float32),
)


def _fin_body(agg_ref, dinv_ref, b_ref, fcw_ref, fcb_ref, h_ref, o_ref):
    h = (jnp.concatenate([agg_ref[0], agg_ref[1]], axis=1) * dinv_ref[...]
         + b_ref[...])
    h_ref[...] = h
    o_ref[...] = lax.dot_general(
        h, fcw_ref[...], (((1,), (1,)), ((), ())),
        preferred_element_type=jnp.float32) + fcb_ref[...]


_fin_call = pl.pallas_call(
    _fin_body,
    grid=(N // RB,),
    in_specs=[
        pl.BlockSpec((NC, RB, HF), lambda i: (0, i, 0)),
        pl.BlockSpec((RB, 1), lambda i: (i, 0)),
        pl.BlockSpec((1, F), lambda i: (0, 0)),
        pl.BlockSpec((NCLS, F), lambda i: (0, 0)),
        pl.BlockSpec((1, NCLS), lambda i: (0, 0)),
    ],
    out_specs=[
        pl.BlockSpec((RB, F), lambda i: (i, 0)),
        pl.BlockSpec((RB, NCLS), lambda i: (i, 0)),
    ],
    out_shape=[
        jax.ShapeDtypeStruct((N, F), jnp.float32),
        jax.ShapeDtypeStruct((N, NCLS), jnp.float32),
    ],
)


def kernel(x, edge_index, W1, b1, W2, b2, W3, b3, fcW, fcb):
    src = edge_index[0]
    dst = edge_index[1]
    dw = _deg_kernel(dst)
    dinv = _dinv_call(dw)
    hs = _mm1_call(x, W1, dinv).reshape(2 * N, HF)
    agg1 = _scatter_kernel(hs, src, dst).reshape(NC, N, HF)
    hs2 = _mid_call(agg1, dinv, b1.reshape(1, F), W2).reshape(2 * N, HF)
    agg2 = _scatter_kernel(hs2, src, dst).reshape(NC, N, HF)
    hs3 = _mid_call(agg2, dinv, b2.reshape(1, F), W3).reshape(2 * N, HF)
    agg3 = _scatter_kernel(hs3, src, dst).reshape(NC, N, HF)
    h, out = _fin_call(agg3, dinv, b3.reshape(1, F), fcW, fcb.reshape(1, NCLS))
    return (out, h)


# speedup vs baseline: 7.1179x; 7.1179x over previous
import functools

import jax
import jax.numpy as jnp
from jax import lax
from jax.experimental import pallas as pl
from jax.experimental.pallas import tpu as pltpu
from jax.experimental.pallas import tpu_sc as plsc

N = 10000
E = 160000
F = 256
HF = 128
NCLS = 64
NC = 2
NS = 16
ROWS_PER_TILE = N // NS
K = 80
EDGES_PER_TILE = E // NS
DEG_EDGES_PER_TILE = E // (NC * NS)
KD = 40
RB = 2000

_mesh = plsc.VectorSubcoreMesh(core_axis_name="c", subcore_axis_name="s")
_sc_params = pltpu.CompilerParams(use_tc_tiling_on_sc=False)


@functools.partial(
    pl.kernel,
    out_type=jax.ShapeDtypeStruct((NC, N, 16), jnp.float32),
    mesh=_mesh,
    compiler_params=_sc_params,
    scratch_types=[
        pltpu.VMEM_SHARED((N, 16), jnp.float32),
        pltpu.VMEM((ROWS_PER_TILE, 16), jnp.float32),
        pltpu.VMEM((KD, 16), jnp.float32),
        pltpu.VMEM((KD,), jnp.int32),
    ],
)
def _deg_kernel(dst_hbm, out_hbm, cnt_sh, zbuf, ones_v, didx):
    c = lax.axis_index("c")
    s = lax.axis_index("s")

    def fill_zero(i, carry):
        zbuf[i, :] = jnp.zeros((16,), jnp.float32)
        return carry

    lax.fori_loop(0, ROWS_PER_TILE, fill_zero, 0)

    def fill_one(i, carry):
        ones_v[i, :] = jnp.ones((16,), jnp.float32)
        return carry

    lax.fori_loop(0, KD, fill_one, 0)

    row0 = s * ROWS_PER_TILE
    pltpu.sync_copy(zbuf, cnt_sh.at[pl.ds(row0, ROWS_PER_TILE)])
    plsc.subcore_barrier()

    base = (c * NS + s) * DEG_EDGES_PER_TILE

    def body(i, carry):
        pltpu.sync_copy(dst_hbm.at[pl.ds(base + i * KD, KD)], didx)
        pltpu.sync_copy(ones_v, cnt_sh.at[didx], add=True)
        return carry

    lax.fori_loop(0, DEG_EDGES_PER_TILE // KD, body, 0)
    plsc.subcore_barrier()

    pltpu.sync_copy(cnt_sh.at[pl.ds(row0, ROWS_PER_TILE)],
                    out_hbm.at[c, pl.ds(row0, ROWS_PER_TILE)])


@functools.partial(
    pl.kernel,
    out_type=jax.ShapeDtypeStruct((2 * N, HF), jnp.float32),
    mesh=_mesh,
    compiler_params=_sc_params,
    scratch_types=[
        pltpu.VMEM_SHARED((N, HF), jnp.float32),
        pltpu.VMEM((K,), jnp.int32),
        pltpu.VMEM((K,), jnp.int32),
        pltpu.VMEM((K,), jnp.int32),
        pltpu.VMEM((K, HF), jnp.float32),
        pltpu.SemaphoreType.DMA,
    ],
)
def _scatter_kernel(hs_hbm, src_hbm, dst_hbm, out_hbm,
                    acc_sh, sidx, gidx, didx, gbuf, sem):
    c = lax.axis_index("c")
    s = lax.axis_index("s")
    row0 = s * ROWS_PER_TILE
    half = c * N

    pltpu.sync_copy(hs_hbm.at[pl.ds(half + row0, ROWS_PER_TILE)],
                    acc_sh.at[pl.ds(row0, ROWS_PER_TILE)])
    plsc.subcore_barrier()

    ebase = s * EDGES_PER_TILE

    def body(i, carry):
        b = ebase + i * K
        pltpu.sync_copy(src_hbm.at[pl.ds(b, K)], sidx)
        pltpu.sync_copy(dst_hbm.at[pl.ds(b, K)], didx)
        for j in range(K // 16):
            gidx[pl.ds(j * 16, 16)] = sidx[pl.ds(j * 16, 16)] + half
        pltpu.async_copy(hs_hbm.at[gidx], gbuf, sem).wait()
        pltpu.sync_copy(gbuf, acc_sh.at[didx], add=True)
        return carry

    lax.fori_loop(0, EDGES_PER_TILE // K, body, 0)
    plsc.subcore_barrier()

    pltpu.sync_copy(acc_sh.at[pl.ds(row0, ROWS_PER_TILE)],
                    out_hbm.at[pl.ds(half + row0, ROWS_PER_TILE)])


def _dinv_body(dw_ref, o_ref):
    deg = dw_ref[0][:, 0:1] + dw_ref[1][:, 0:1] + 1.0
    o_ref[...] = lax.rsqrt(deg)


_dinv_call = pl.pallas_call(
    _dinv_body,
    in_specs=[pl.BlockSpec((NC, N, 16), lambda: (0, 0, 0))],
    out_specs=pl.BlockSpec((N, 1), lambda: (0, 0)),
    out_shape=jax.ShapeDtypeStruct((N, 1), jnp.float32),
)


def _mm1_body(x_ref, w_ref, dinv_ref, o_ref):
    y = jnp.dot(x_ref[...], w_ref[...], preferred_element_type=jnp.float32)
    y = y * dinv_ref[...]
    o_ref[0] = y[:, :HF]
    o_ref[1] = y[:, HF:]


_mm1_call = pl.pallas_call(
    _mm1_body,
    grid=(N // RB,),
    in_specs=[
        pl.BlockSpec((RB, F), lambda i: (i, 0)),
        pl.BlockSpec((F, F), lambda i: (0, 0)),
        pl.BlockSpec((RB, 1), lambda i: (i, 0)),
    ],
    out_specs=pl.BlockSpec((NC, RB, HF), lambda i: (0, i, 0)),
    out_shape=jax.ShapeDtypeStruct((NC, N, HF), jnp.float32),
)


def _mid_body(agg_ref, dinv_ref, b_ref, w_ref, o_ref):
    dinv = dinv_ref[...]
    h = jnp.concatenate([agg_ref[0], agg_ref[1]], axis=1) * dinv + b_ref[...]
    y = jnp.dot(h, w_ref[...], preferred_element_type=jnp.float32) * dinv
    o_ref[0] = y[:, :HF]
    o_ref[1] = y[:, HF:]


_mid_call = pl.pallas_call(
    _mid_body,
    grid=(N // RB,),
    in_specs=[
        pl.BlockSpec((NC, RB, HF), lambda i: (0, i, 0)),
        pl.BlockSpec((RB, 1), lambda i: (i, 0)),
        pl.BlockSpec((1, F), lambda i: (0, 0)),
        pl.BlockSpec((F, F), lambda i: (0, 0)),
    ],
    out_specs=pl.BlockSpec((NC, RB, HF), lambda i: (0, i, 0)),
    out_shape=jax.ShapeDtypeStruct((NC, N, HF), jnp.float32),
)


def _fin_body(agg_ref, dinv_ref, b_ref, fcw_ref, fcb_ref, h_ref, o_ref):
    h = (jnp.concatenate([agg_ref[0], agg_ref[1]], axis=1) * dinv_ref[...]
         + b_ref[...])
    h_ref[...] = h
    o_ref[...] = lax.dot_general(
        h, fcw_ref[...], (((1,), (1,)), ((), ())),
        preferred_element_type=jnp.float32) + fcb_ref[...]


_fin_call = pl.pallas_call(
    _fin_body,
    grid=(N // RB,),
    in_specs=[
        pl.BlockSpec((NC, RB, HF), lambda i: (0, i, 0)),
        pl.BlockSpec((RB, 1), lambda i: (i, 0)),
        pl.BlockSpec((1, F), lambda i: (0, 0)),
        pl.BlockSpec((NCLS, F), lambda i: (0, 0)),
        pl.BlockSpec((1, NCLS), lambda i: (0, 0)),
    ],
    out_specs=[
        pl.BlockSpec((RB, F), lambda i: (i, 0)),
        pl.BlockSpec((RB, NCLS), lambda i: (i, 0)),
    ],
    out_shape=[
        jax.ShapeDtypeStruct((N, F), jnp.float32),
        jax.ShapeDtypeStruct((N, NCLS), jnp.float32),
    ],
)


def kernel(x, edge_index, W1, b1, W2, b2, W3, b3, fcW, fcb):
    src = edge_index[0]
    dst = edge_index[1]
    dw = _deg_kernel(dst)
    dinv = _dinv_call(dw)
    hs = _mm1_call(x, W1, dinv).reshape(2 * N, HF)
    agg1 = _scatter_kernel(hs, src, dst).reshape(NC, N, HF)
    hs2 = _mid_call(agg1, dinv, b1.reshape(1, F), W2).reshape(2 * N, HF)
    agg2 = _scatter_kernel(hs2, src, dst).reshape(NC, N, HF)
    hs3 = _mid_call(agg2, dinv, b2.reshape(1, F), W3).reshape(2 * N, HF)
    agg3 = _scatter_kernel(hs3, src, dst).reshape(NC, N, HF)
    h, out = _fin_call(agg3, dinv, b3.reshape(1, F), fcW, fcb.reshape(1, NCLS))
    return (out, h)


# speedup vs baseline: 13.4929x; 1.8956x over previous
import functools

import jax
import jax.numpy as jnp
from jax import lax
from jax.experimental import pallas as pl
from jax.experimental.pallas import tpu as pltpu
from jax.experimental.pallas import tpu_sc as plsc

N = 10000
E = 160000
F = 256
HF = 128
NCLS = 64
NC = 2
NS = 16
ROWS_PER_TILE = N // NS
K = 40
EDGES_PER_TILE = E // NS
DEG_EDGES_PER_TILE = E // (NC * NS)
KD = 40
NCH = EDGES_PER_TILE // K
NCHD = DEG_EDGES_PER_TILE // KD
NBUF = 5
GROUPS = NCH // NBUF
RB = 2000

_mesh = plsc.VectorSubcoreMesh(core_axis_name="c", subcore_axis_name="s")
_sc_params = pltpu.CompilerParams(use_tc_tiling_on_sc=False)


@functools.partial(
    pl.kernel,
    out_type=jax.ShapeDtypeStruct((NC, N, 16), jnp.float32),
    mesh=_mesh,
    compiler_params=_sc_params,
    scratch_types=[
        pltpu.VMEM_SHARED((N, 16), jnp.float32),
        pltpu.VMEM((25, 16), jnp.float32),
        pltpu.VMEM((KD, 16), jnp.float32),
        pltpu.VMEM((KD,), jnp.int32),
    ],
)
def _deg_kernel(dst_hbm, out_hbm, cnt_sh, zbuf, ones_v, didx):
    c = lax.axis_index("c")
    s = lax.axis_index("s")
    w = c * NS + s

    def fill_zero(i, carry):
        zbuf[i, :] = jnp.zeros((16,), jnp.float32)
        return carry

    lax.fori_loop(0, 25, fill_zero, 0)

    def fill_one(i, carry):
        ones_v[i, :] = jnp.ones((16,), jnp.float32)
        return carry

    lax.fori_loop(0, KD, fill_one, 0)

    row0 = s * ROWS_PER_TILE

    def zero_out(i, carry):
        pltpu.sync_copy(zbuf, cnt_sh.at[pl.ds(row0 + i * 25, 25)])
        return carry

    lax.fori_loop(0, ROWS_PER_TILE // 25, zero_out, 0)
    plsc.subcore_barrier()

    base = w * DEG_EDGES_PER_TILE

    def fire(i, carry):
        pltpu.sync_copy(dst_hbm.at[pl.ds(base + i * KD, KD)], didx)
        pltpu.sync_copy(ones_v, cnt_sh.at[didx], add=True)
        return carry

    lax.fori_loop(0, NCHD, fire, 0)
    plsc.subcore_barrier()

    pltpu.sync_copy(cnt_sh.at[pl.ds(row0, ROWS_PER_TILE)],
                    out_hbm.at[c, pl.ds(row0, ROWS_PER_TILE)])


@functools.partial(
    pl.kernel,
    out_type=jax.ShapeDtypeStruct((2 * N, HF), jnp.float32),
    mesh=_mesh,
    compiler_params=_sc_params,
    scratch_types=[
        pltpu.VMEM_SHARED((N, HF), jnp.float32),
        *([pltpu.VMEM((K,), jnp.int32)] * NBUF),
        *([pltpu.VMEM((K,), jnp.int32)] * NBUF),
        *([pltpu.VMEM((K,), jnp.int32)] * NBUF),
        *([pltpu.VMEM((K, HF), jnp.float32)] * NBUF),
        *([pltpu.SemaphoreType.DMA] * (3 * NBUF)),
    ],
)
def _scatter_kernel(hs_hbm, src_hbm, dst_hbm, out_hbm, acc_sh, *ring):
    sis = ring[:NBUF]
    gis = ring[NBUF:2 * NBUF]
    dis = ring[2 * NBUF:3 * NBUF]
    gbufs = ring[3 * NBUF:4 * NBUF]
    isems = ring[4 * NBUF:5 * NBUF]
    gsems = ring[5 * NBUF:6 * NBUF]
    ssems = ring[6 * NBUF:]
    c = lax.axis_index("c")
    s = lax.axis_index("s")
    row0 = s * ROWS_PER_TILE
    half = c * N

    pltpu.sync_copy(hs_hbm.at[pl.ds(half + row0, ROWS_PER_TILE)],
                    acc_sh.at[pl.ds(row0, ROWS_PER_TILE)])
    ebase = s * EDGES_PER_TILE
    windows = sorted({min(j * 16, K - 16) for j in range((K + 15) // 16)})
    for b in range(NBUF):
        pltpu.async_copy(src_hbm.at[pl.ds(ebase + b * K, K)], sis[b], isems[b])
        pltpu.async_copy(dst_hbm.at[pl.ds(ebase + b * K, K)], dis[b], isems[b])
    plsc.subcore_barrier()

    def outer(t, carry):
        gd = []
        for b in range(NBUF):
            i = t * NBUF + b
            pltpu.make_async_copy(src_hbm.at[pl.ds(ebase + i * K, K)],
                                  sis[b], isems[b]).wait()
            pltpu.make_async_copy(dst_hbm.at[pl.ds(ebase + i * K, K)],
                                  dis[b], isems[b]).wait()
            for w0 in windows:
                gis[b][pl.ds(w0, 16)] = sis[b][pl.ds(w0, 16)] + half
            gd.append(pltpu.async_copy(hs_hbm.at[gis[b]], gbufs[b], gsems[b]))
        sd = []
        for b in range(NBUF):
            gd[b].wait()
            sd.append(pltpu.async_copy(gbufs[b], acc_sh.at[dis[b]],
                                       ssems[b], add=True))
        for b in range(NBUF):
            sd[b].wait()

            @pl.when(t + 1 < GROUPS)
            def _():
                nb = ebase + ((t + 1) * NBUF + b) * K
                pltpu.async_copy(src_hbm.at[pl.ds(nb, K)], sis[b], isems[b])
                pltpu.async_copy(dst_hbm.at[pl.ds(nb, K)], dis[b], isems[b])
        return carry

    lax.fori_loop(0, GROUPS, outer, 0)
    plsc.subcore_barrier()

    pltpu.sync_copy(acc_sh.at[pl.ds(row0, ROWS_PER_TILE)],
                    out_hbm.at[pl.ds(half + row0, ROWS_PER_TILE)])


def _dinv_body(dw_ref, o_ref):
    deg = dw_ref[0][:, 0:1] + dw_ref[1][:, 0:1] + 1.0
    o_ref[...] = lax.rsqrt(deg)


_dinv_call = pl.pallas_call(
    _dinv_body,
    in_specs=[pl.BlockSpec((NC, N, 16), lambda: (0, 0, 0))],
    out_specs=pl.BlockSpec((N, 1), lambda: (0, 0)),
    out_shape=jax.ShapeDtypeStruct((N, 1), jnp.float32),
)


def _mm1_body(x_ref, w_ref, dinv_ref, o_ref):
    y = jnp.dot(x_ref[...], w_ref[...], preferred_element_type=jnp.float32)
    y = y * dinv_ref[...]
    o_ref[0] = y[:, :HF]
    o_ref[1] = y[:, HF:]


_mm1_call = pl.pallas_call(
    _mm1_body,
    grid=(N // RB,),
    in_specs=[
        pl.BlockSpec((RB, F), lambda i: (i, 0)),
        pl.BlockSpec((F, F), lambda i: (0, 0)),
        pl.BlockSpec((RB, 1), lambda i: (i, 0)),
    ],
    out_specs=pl.BlockSpec((NC, RB, HF), lambda i: (0, i, 0)),
    out_shape=jax.ShapeDtypeStruct((NC, N, HF), jnp.float32),
)


def _mid_body(agg_ref, dinv_ref, b_ref, w_ref, o_ref):
    dinv = dinv_ref[...]
    h = jnp.concatenate([agg_ref[0], agg_ref[1]], axis=1) * dinv + b_ref[...]
    y = jnp.dot(h, w_ref[...], preferred_element_type=jnp.float32) * dinv
    o_ref[0] = y[:, :HF]
    o_ref[1] = y[:, HF:]


_mid_call = pl.pallas_call(
    _mid_body,
    grid=(N // RB,),
    in_specs=[
        pl.BlockSpec((NC, RB, HF), lambda i: (0, i, 0)),
        pl.BlockSpec((RB, 1), lambda i: (i, 0)),
        pl.BlockSpec((1, F), lambda i: (0, 0)),
        pl.BlockSpec((F, F), lambda i: (0, 0)),
    ],
    out_specs=pl.BlockSpec((NC, RB, HF), lambda i: (0, i, 0)),
    out_shape=jax.ShapeDtypeStruct((NC, N, HF), jnp.float32),
)


def _fin_body(agg_ref, dinv_ref, b_ref, fcw_ref, fcb_ref, h_ref, o_ref):
    h = (jnp.concatenate([agg_ref[0], agg_ref[1]], axis=1) * dinv_ref[...]
         + b_ref[...])
    h_ref[...] = h
    o_ref[...] = lax.dot_general(
        h, fcw_ref[...], (((1,), (1,)), ((), ())),
        preferred_element_type=jnp.float32) + fcb_ref[...]


_fin_call = pl.pallas_call(
    _fin_body,
    grid=(N // RB,),
    in_specs=[
        pl.BlockSpec((NC, RB, HF), lambda i: (0, i, 0)),
        pl.BlockSpec((RB, 1), lambda i: (i, 0)),
        pl.BlockSpec((1, F), lambda i: (0, 0)),
        pl.BlockSpec((NCLS, F), lambda i: (0, 0)),
        pl.BlockSpec((1, NCLS), lambda i: (0, 0)),
    ],
    out_specs=[
        pl.BlockSpec((RB, F), lambda i: (i, 0)),
        pl.BlockSpec((RB, NCLS), lambda i: (i, 0)),
    ],
    out_shape=[
        jax.ShapeDtypeStruct((N, F), jnp.float32),
        jax.ShapeDtypeStruct((N, NCLS), jnp.float32),
    ],
)


def kernel(x, edge_index, W1, b1, W2, b2, W3, b3, fcW, fcb):
    src = edge_index[0]
    dst = edge_index[1]
    dw = _deg_kernel(dst)
    dinv = _dinv_call(dw)
    hs = _mm1_call(x, W1, dinv).reshape(2 * N, HF)
    agg1 = _scatter_kernel(hs, src, dst).reshape(NC, N, HF)
    hs2 = _mid_call(agg1, dinv, b1.reshape(1, F), W2).reshape(2 * N, HF)
    agg2 = _scatter_kernel(hs2, src, dst).reshape(NC, N, HF)
    hs3 = _mid_call(agg2, dinv, b2.reshape(1, F), W3).reshape(2 * N, HF)
    agg3 = _scatter_kernel(hs3, src, dst).reshape(NC, N, HF)
    h, out = _fin_call(agg3, dinv, b3.reshape(1, F), fcW, fcb.reshape(1, NCLS))
    return (out, h)


# speedup vs baseline: 14.9861x; 1.1107x over previous
import functools

import jax
import jax.numpy as jnp
from jax import lax
from jax.experimental import pallas as pl
from jax.experimental.pallas import tpu as pltpu
from jax.experimental.pallas import tpu_sc as plsc

N = 10000
E = 160000
F = 256
HF = 128
NCLS = 64
NC = 2
NS = 16
ROWS_PER_TILE = N // NS
K = 40
EDGES_PER_TILE = E // NS
DEG_EDGES_PER_TILE = E // (NC * NS)
KD = 40
NCH = EDGES_PER_TILE // K
NCHD = DEG_EDGES_PER_TILE // KD
NBUF = 5
GROUPS = NCH // NBUF
RB = 2000

_mesh = plsc.VectorSubcoreMesh(core_axis_name="c", subcore_axis_name="s")
_sc_params = pltpu.CompilerParams(use_tc_tiling_on_sc=False)


@functools.partial(
    pl.kernel,
    out_type=jax.ShapeDtypeStruct((NC, N, 16), jnp.float32),
    mesh=_mesh,
    compiler_params=_sc_params,
    scratch_types=[
        pltpu.VMEM_SHARED((N, 16), jnp.float32),
        pltpu.VMEM((25, 16), jnp.float32),
        pltpu.VMEM((KD, 16), jnp.float32),
        pltpu.VMEM((NCHD, KD), jnp.int32),
        pltpu.SemaphoreType.DMA,
    ],
)
def _deg_kernel(dst_hbm, out_hbm, cnt_sh, zbuf, ones_v, didx2, dsem):
    c = lax.axis_index("c")
    s = lax.axis_index("s")
    w = c * NS + s

    def fill_zero(i, carry):
        zbuf[i, :] = jnp.zeros((16,), jnp.float32)
        return carry

    lax.fori_loop(0, 25, fill_zero, 0)

    def fill_one(i, carry):
        ones_v[i, :] = jnp.ones((16,), jnp.float32)
        return carry

    lax.fori_loop(0, KD, fill_one, 0)

    row0 = s * ROWS_PER_TILE

    def zero_out(i, carry):
        pltpu.sync_copy(zbuf, cnt_sh.at[pl.ds(row0 + i * 25, 25)])
        return carry

    lax.fori_loop(0, ROWS_PER_TILE // 25, zero_out, 0)
    plsc.subcore_barrier()

    base = w * DEG_EDGES_PER_TILE

    def ldx(i, carry):
        pltpu.async_copy(dst_hbm.at[pl.ds(base + i * KD, KD)],
                         didx2.at[i], dsem)
        return carry

    lax.fori_loop(0, NCHD, ldx, 0)

    def ldx_drain(i, carry):
        pltpu.make_async_copy(dst_hbm.at[pl.ds(base, KD)],
                              didx2.at[0], dsem).wait()
        return carry

    lax.fori_loop(0, NCHD, ldx_drain, 0)

    def fire(i, carry):
        pltpu.async_copy(ones_v, cnt_sh.at[didx2.at[i]], dsem, add=True)
        return carry

    lax.fori_loop(0, NCHD, fire, 0)

    def drain(i, carry):
        pltpu.make_async_copy(ones_v, cnt_sh.at[didx2.at[0]], dsem).wait()
        return carry

    lax.fori_loop(0, NCHD, drain, 0)
    plsc.subcore_barrier()

    pltpu.sync_copy(cnt_sh.at[pl.ds(row0, ROWS_PER_TILE)],
                    out_hbm.at[c, pl.ds(row0, ROWS_PER_TILE)])


@functools.partial(
    pl.kernel,
    out_type=jax.ShapeDtypeStruct((2 * N, HF), jnp.float32),
    mesh=_mesh,
    compiler_params=_sc_params,
    scratch_types=[
        pltpu.VMEM_SHARED((N, HF), jnp.float32),
        *([pltpu.VMEM((K,), jnp.int32)] * NBUF),
        *([pltpu.VMEM((K,), jnp.int32)] * NBUF),
        *([pltpu.VMEM((K,), jnp.int32)] * NBUF),
        *([pltpu.VMEM((K, HF), jnp.float32)] * NBUF),
        *([pltpu.SemaphoreType.DMA] * (3 * NBUF)),
    ],
)
def _scatter_kernel(hs_hbm, src_hbm, dst_hbm, out_hbm, acc_sh, *ring):
    sis = ring[:NBUF]
    gis = ring[NBUF:2 * NBUF]
    dis = ring[2 * NBUF:3 * NBUF]
    gbufs = ring[3 * NBUF:4 * NBUF]
    isems = ring[4 * NBUF:5 * NBUF]
    gsems = ring[5 * NBUF:6 * NBUF]
    ssems = ring[6 * NBUF:]
    c = lax.axis_index("c")
    s = lax.axis_index("s")
    row0 = s * ROWS_PER_TILE
    half = c * N

    pltpu.sync_copy(hs_hbm.at[pl.ds(half + row0, ROWS_PER_TILE)],
                    acc_sh.at[pl.ds(row0, ROWS_PER_TILE)])
    ebase = s * EDGES_PER_TILE
    windows = sorted({min(j * 16, K - 16) for j in range((K + 15) // 16)})
    for b in range(NBUF):
        pltpu.async_copy(src_hbm.at[pl.ds(ebase + b * K, K)], sis[b], isems[b])
        pltpu.async_copy(dst_hbm.at[pl.ds(ebase + b * K, K)], dis[b], isems[b])
    plsc.subcore_barrier()

    def outer(t, carry):
        gd = []
        for b in range(NBUF):
            i = t * NBUF + b
            pltpu.make_async_copy(src_hbm.at[pl.ds(ebase + i * K, K)],
                                  sis[b], isems[b]).wait()
            pltpu.make_async_copy(dst_hbm.at[pl.ds(ebase + i * K, K)],
                                  dis[b], isems[b]).wait()
            for w0 in windows:
                gis[b][pl.ds(w0, 16)] = sis[b][pl.ds(w0, 16)] + half
            gd.append(pltpu.async_copy(hs_hbm.at[gis[b]], gbufs[b], gsems[b]))
        sd = []
        for b in range(NBUF):
            gd[b].wait()
            sd.append(pltpu.async_copy(gbufs[b], acc_sh.at[dis[b]],
                                       ssems[b], add=True))
        for b in range(NBUF):
            sd[b].wait()

            @pl.when(t + 1 < GROUPS)
            def _():
                nb = ebase + ((t + 1) * NBUF + b) * K
                pltpu.async_copy(src_hbm.at[pl.ds(nb, K)], sis[b], isems[b])
                pltpu.async_copy(dst_hbm.at[pl.ds(nb, K)], dis[b], isems[b])
        return carry

    lax.fori_loop(0, GROUPS, outer, 0)
    plsc.subcore_barrier()

    pltpu.sync_copy(acc_sh.at[pl.ds(row0, ROWS_PER_TILE)],
                    out_hbm.at[pl.ds(half + row0, ROWS_PER_TILE)])


def _dinv_body(dw_ref, o_ref):
    deg = dw_ref[0][:, 0:1] + dw_ref[1][:, 0:1] + 1.0
    o_ref[...] = lax.rsqrt(deg)


_dinv_call = pl.pallas_call(
    _dinv_body,
    in_specs=[pl.BlockSpec((NC, N, 16), lambda: (0, 0, 0))],
    out_specs=pl.BlockSpec((N, 1), lambda: (0, 0)),
    out_shape=jax.ShapeDtypeStruct((N, 1), jnp.float32),
)


def _mm1_body(x_ref, w_ref, dinv_ref, o_ref):
    y = jnp.dot(x_ref[...], w_ref[...], preferred_element_type=jnp.float32)
    y = y * dinv_ref[...]
    o_ref[0] = y[:, :HF]
    o_ref[1] = y[:, HF:]


_mm1_call = pl.pallas_call(
    _mm1_body,
    grid=(N // RB,),
    in_specs=[
        pl.BlockSpec((RB, F), lambda i: (i, 0)),
        pl.BlockSpec((F, F), lambda i: (0, 0)),
        pl.BlockSpec((RB, 1), lambda i: (i, 0)),
    ],
    out_specs=pl.BlockSpec((NC, RB, HF), lambda i: (0, i, 0)),
    out_shape=jax.ShapeDtypeStruct((NC, N, HF), jnp.float32),
)


def _mid_body(agg_ref, dinv_ref, b_ref, w_ref, o_ref):
    dinv = dinv_ref[...]
    h = jnp.concatenate([agg_ref[0], agg_ref[1]], axis=1) * dinv + b_ref[...]
    y = jnp.dot(h, w_ref[...], preferred_element_type=jnp.float32) * dinv
    o_ref[0] = y[:, :HF]
    o_ref[1] = y[:, HF:]


_mid_call = pl.pallas_call(
    _mid_body,
    grid=(N // RB,),
    in_specs=[
        pl.BlockSpec((NC, RB, HF), lambda i: (0, i, 0)),
        pl.BlockSpec((RB, 1), lambda i: (i, 0)),
        pl.BlockSpec((1, F), lambda i: (0, 0)),
        pl.BlockSpec((F, F), lambda i: (0, 0)),
    ],
    out_specs=pl.BlockSpec((NC, RB, HF), lambda i: (0, i, 0)),
    out_shape=jax.ShapeDtypeStruct((NC, N, HF), jnp.float32),
)


def _fin_body(agg_ref, dinv_ref, b_ref, fcw_ref, fcb_ref, h_ref, o_ref):
    h = (jnp.concatenate([agg_ref[0], agg_ref[1]], axis=1) * dinv_ref[...]
         + b_ref[...])
    h_ref[...] = h
    o_ref[...] = lax.dot_general(
        h, fcw_ref[...], (((1,), (1,)), ((), ())),
        preferred_element_type=jnp.float32) + fcb_ref[...]


_fin_call = pl.pallas_call(
    _fin_body,
    grid=(N // RB,),
    in_specs=[
        pl.BlockSpec((NC, RB, HF), lambda i: (0, i, 0)),
        pl.BlockSpec((RB, 1), lambda i: (i, 0)),
        pl.BlockSpec((1, F), lambda i: (0, 0)),
        pl.BlockSpec((NCLS, F), lambda i: (0, 0)),
        pl.BlockSpec((1, NCLS), lambda i: (0, 0)),
    ],
    out_specs=[
        pl.BlockSpec((RB, F), lambda i: (i, 0)),
        pl.BlockSpec((RB, NCLS), lambda i: (i, 0)),
    ],
    out_shape=[
        jax.ShapeDtypeStruct((N, F), jnp.float32),
        jax.ShapeDtypeStruct((N, NCLS), jnp.float32),
    ],
)


def kernel(x, edge_index, W1, b1, W2, b2, W3, b3, fcW, fcb):
    src = edge_index[0]
    dst = edge_index[1]
    dw = _deg_kernel(dst)
    dinv = _dinv_call(dw)
    hs = _mm1_call(x, W1, dinv).reshape(2 * N, HF)
    agg1 = _scatter_kernel(hs, src, dst).reshape(NC, N, HF)
    hs2 = _mid_call(agg1, dinv, b1.reshape(1, F), W2).reshape(2 * N, HF)
    agg2 = _scatter_kernel(hs2, src, dst).reshape(NC, N, HF)
    hs3 = _mid_call(agg2, dinv, b2.reshape(1, F), W3).reshape(2 * N, HF)
    agg3 = _scatter_kernel(hs3, src, dst).reshape(NC, N, HF)
    h, out = _fin_call(agg3, dinv, b3.reshape(1, F), fcW, fcb.reshape(1, NCLS))
    return (out, h)


# speedup vs baseline: 17.1654x; 1.1454x over previous
import functools

import jax
import jax.numpy as jnp
from jax import lax
from jax.experimental import pallas as pl
from jax.experimental.pallas import tpu as pltpu
from jax.experimental.pallas import tpu_sc as plsc

N = 10000
E = 160000
F = 256
HF = 128
NCLS = 64
NC = 2
NS = 16
ROWS_PER_TILE = N // NS
K = 40
EDGES_PER_TILE = E // NS
DEG_EDGES_PER_TILE = E // (NC * NS)
KD = 40
NCH = EDGES_PER_TILE // K
NCHD = DEG_EDGES_PER_TILE // KD
NBUF = 5
GROUPS = NCH // NBUF
RB = 2000

_mesh = plsc.VectorSubcoreMesh(core_axis_name="c", subcore_axis_name="s")
_sc_params = pltpu.CompilerParams(use_tc_tiling_on_sc=False)


@functools.partial(
    pl.kernel,
    out_type=jax.ShapeDtypeStruct((NC, N, 16), jnp.float32),
    mesh=_mesh,
    compiler_params=_sc_params,
    scratch_types=[
        pltpu.VMEM_SHARED((N, 16), jnp.float32),
        pltpu.VMEM((25, 16), jnp.float32),
        pltpu.VMEM((KD, 16), jnp.float32),
        pltpu.VMEM((NCHD, KD), jnp.int32),
        pltpu.SemaphoreType.DMA,
    ],
)
def _deg_kernel(dst_hbm, out_hbm, cnt_sh, zbuf, ones_v, didx2, dsem):
    c = lax.axis_index("c")
    s = lax.axis_index("s")
    w = c * NS + s

    def fill_zero(i, carry):
        zbuf[i, :] = jnp.zeros((16,), jnp.float32)
        return carry

    lax.fori_loop(0, 25, fill_zero, 0)

    def fill_one(i, carry):
        ones_v[i, :] = jnp.ones((16,), jnp.float32)
        return carry

    lax.fori_loop(0, KD, fill_one, 0)

    row0 = s * ROWS_PER_TILE

    def zero_out(i, carry):
        pltpu.sync_copy(zbuf, cnt_sh.at[pl.ds(row0 + i * 25, 25)])
        return carry

    lax.fori_loop(0, ROWS_PER_TILE // 25, zero_out, 0)
    plsc.subcore_barrier()

    base = w * DEG_EDGES_PER_TILE

    def ldx(i, carry):
        pltpu.async_copy(dst_hbm.at[pl.ds(base + i * KD, KD)],
                         didx2.at[i], dsem)
        return carry

    lax.fori_loop(0, NCHD, ldx, 0)

    def ldx_drain(i, carry):
        pltpu.make_async_copy(dst_hbm.at[pl.ds(base, KD)],
                              didx2.at[0], dsem).wait()
        return carry

    lax.fori_loop(0, NCHD, ldx_drain, 0)

    def fire(i, carry):
        pltpu.async_copy(ones_v, cnt_sh.at[didx2.at[i]], dsem, add=True)
        return carry

    lax.fori_loop(0, NCHD, fire, 0)

    def drain(i, carry):
        pltpu.make_async_copy(ones_v, cnt_sh.at[didx2.at[0]], dsem).wait()
        return carry

    lax.fori_loop(0, NCHD, drain, 0)
    plsc.subcore_barrier()

    pltpu.sync_copy(cnt_sh.at[pl.ds(row0, ROWS_PER_TILE)],
                    out_hbm.at[c, pl.ds(row0, ROWS_PER_TILE)])


@functools.partial(
    pl.kernel,
    out_type=jax.ShapeDtypeStruct((2 * N, HF), jnp.float32),
    mesh=_mesh,
    compiler_params=_sc_params,
    scratch_types=[
        pltpu.VMEM_SHARED((N, HF), jnp.float32),
        *([pltpu.VMEM((K,), jnp.int32)] * (2 * NBUF)),
        *([pltpu.VMEM((K,), jnp.int32)] * (2 * NBUF)),
        *([pltpu.VMEM((K,), jnp.int32)] * (2 * NBUF)),
        *([pltpu.VMEM((K, HF), jnp.float32)] * NBUF),
        *([pltpu.SemaphoreType.DMA] * (4 * NBUF)),
    ],
)
def _scatter_kernel(hs_hbm, src_hbm, dst_hbm, out_hbm, acc_sh, *ring):
    IB = 2 * NBUF
    sis = ring[:IB]
    gis = ring[IB:2 * IB]
    dis = ring[2 * IB:3 * IB]
    gbufs = ring[3 * IB:3 * IB + NBUF]
    isems = ring[3 * IB + NBUF:4 * IB + NBUF]
    gsems = ring[4 * IB + NBUF:4 * IB + 2 * NBUF]
    ssems = ring[4 * IB + 2 * NBUF:]
    c = lax.axis_index("c")
    s = lax.axis_index("s")
    row0 = s * ROWS_PER_TILE
    half = c * N

    pltpu.sync_copy(hs_hbm.at[pl.ds(half + row0, ROWS_PER_TILE)],
                    acc_sh.at[pl.ds(row0, ROWS_PER_TILE)])
    ebase = s * EDGES_PER_TILE
    windows = sorted({min(j * 16, K - 16) for j in range((K + 15) // 16)})

    def ld_idx(chunk, q):
        nb = ebase + chunk * K
        pltpu.async_copy(src_hbm.at[pl.ds(nb, K)], sis[q], isems[q])
        pltpu.async_copy(dst_hbm.at[pl.ds(nb, K)], dis[q], isems[q])

    def wait_idx(q):
        pltpu.make_async_copy(src_hbm.at[pl.ds(ebase, K)],
                              sis[q], isems[q]).wait()
        pltpu.make_async_copy(dst_hbm.at[pl.ds(ebase, K)],
                              dis[q], isems[q]).wait()

    def wait_scatter(b, q):
        pltpu.make_async_copy(gbufs[b], acc_sh.at[dis[q]], ssems[b]).wait()

    for q in range(IB):
        ld_idx(q, q)
    plsc.subcore_barrier()

    def outer(t, carry):
        for p in (0, 1):
            gd = []
            for b in range(NBUF):
                q = p * NBUF + b
                wait_idx(q)
                for w0 in windows:
                    gis[q][pl.ds(w0, 16)] = sis[q][pl.ds(w0, 16)] + half
                pq = (1 - p) * NBUF + b
                if p == 1:
                    wait_scatter(b, pq)
                else:
                    @pl.when(t > 0)
                    def _():
                        wait_scatter(b, pq)
                gd.append(pltpu.async_copy(hs_hbm.at[gis[q]], gbufs[b],
                                           gsems[b]))
            for b in range(NBUF):
                q = p * NBUF + b
                gd[b].wait()
                pltpu.async_copy(gbufs[b], acc_sh.at[dis[q]], ssems[b],
                                 add=True)
                pq = (1 - p) * NBUF + b
                nxt = (2 * t + p + 1) * NBUF + b
                if p == 0:
                    @pl.when(jnp.logical_and(t > 0, nxt < NCH))
                    def _():
                        ld_idx(nxt, pq)
                else:
                    @pl.when(nxt < NCH)
                    def _():
                        ld_idx(nxt, pq)
        return carry

    lax.fori_loop(0, NCH // (2 * NBUF), outer, 0)
    for b in range(NBUF):
        wait_scatter(b, NBUF + b)
    plsc.subcore_barrier()

    pltpu.sync_copy(acc_sh.at[pl.ds(row0, ROWS_PER_TILE)],
                    out_hbm.at[pl.ds(half + row0, ROWS_PER_TILE)])


def _dinv_body(dw_ref, o_ref):
    deg = dw_ref[0][:, 0:1] + dw_ref[1][:, 0:1] + 1.0
    o_ref[...] = lax.rsqrt(deg)


_dinv_call = pl.pallas_call(
    _dinv_body,
    in_specs=[pl.BlockSpec((NC, N, 16), lambda: (0, 0, 0))],
    out_specs=pl.BlockSpec((N, 1), lambda: (0, 0)),
    out_shape=jax.ShapeDtypeStruct((N, 1), jnp.float32),
)


def _mm1_body(x_ref, w_ref, dinv_ref, o_ref):
    y = jnp.dot(x_ref[...], w_ref[...], preferred_element_type=jnp.float32)
    y = y * dinv_ref[...]
    o_ref[0] = y[:, :HF]
    o_ref[1] = y[:, HF:]


_mm1_call = pl.pallas_call(
    _mm1_body,
    grid=(N // RB,),
    in_specs=[
        pl.BlockSpec((RB, F), lambda i: (i, 0)),
        pl.BlockSpec((F, F), lambda i: (0, 0)),
        pl.BlockSpec((RB, 1), lambda i: (i, 0)),
    ],
    out_specs=pl.BlockSpec((NC, RB, HF), lambda i: (0, i, 0)),
    out_shape=jax.ShapeDtypeStruct((NC, N, HF), jnp.float32),
)


def _mid_body(agg_ref, dinv_ref, b_ref, w_ref, o_ref):
    dinv = dinv_ref[...]
    h = jnp.concatenate([agg_ref[0], agg_ref[1]], axis=1) * dinv + b_ref[...]
    y = jnp.dot(h, w_ref[...], preferred_element_type=jnp.float32) * dinv
    o_ref[0] = y[:, :HF]
    o_ref[1] = y[:, HF:]


_mid_call = pl.pallas_call(
    _mid_body,
    grid=(N // RB,),
    in_specs=[
        pl.BlockSpec((NC, RB, HF), lambda i: (0, i, 0)),
        pl.BlockSpec((RB, 1), lambda i: (i, 0)),
        pl.BlockSpec((1, F), lambda i: (0, 0)),
        pl.BlockSpec((F, F), lambda i: (0, 0)),
    ],
    out_specs=pl.BlockSpec((NC, RB, HF), lambda i: (0, i, 0)),
    out_shape=jax.ShapeDtypeStruct((NC, N, HF), jnp.float32),
)


def _fin_body(agg_ref, dinv_ref, b_ref, fcw_ref, fcb_ref, h_ref, o_ref):
    h = (jnp.concatenate([agg_ref[0], agg_ref[1]], axis=1) * dinv_ref[...]
         + b_ref[...])
    h_ref[...] = h
    o_ref[...] = lax.dot_general(
        h, fcw_ref[...], (((1,), (1,)), ((), ())),
        preferred_element_type=jnp.float32) + fcb_ref[...]


_fin_call = pl.pallas_call(
    _fin_body,
    grid=(N // RB,),
    in_specs=[
        pl.BlockSpec((NC, RB, HF), lambda i: (0, i, 0)),
        pl.BlockSpec((RB, 1), lambda i: (i, 0)),
        pl.BlockSpec((1, F), lambda i: (0, 0)),
        pl.BlockSpec((NCLS, F), lambda i: (0, 0)),
        pl.BlockSpec((1, NCLS), lambda i: (0, 0)),
    ],
    out_specs=[
        pl.BlockSpec((RB, F), lambda i: (i, 0)),
        pl.BlockSpec((RB, NCLS), lambda i: (i, 0)),
    ],
    out_shape=[
        jax.ShapeDtypeStruct((N, F), jnp.float32),
        jax.ShapeDtypeStruct((N, NCLS), jnp.float32),
    ],
)


def kernel(x, edge_index, W1, b1, W2, b2, W3, b3, fcW, fcb):
    src = edge_index[0]
    dst = edge_index[1]
    dw = _deg_kernel(dst)
    dinv = _dinv_call(dw)
    hs = _mm1_call(x, W1, dinv).reshape(2 * N, HF)
    agg1 = _scatter_kernel(hs, src, dst).reshape(NC, N, HF)
    hs2 = _mid_call(agg1, dinv, b1.reshape(1, F), W2).reshape(2 * N, HF)
    agg2 = _scatter_kernel(hs2, src, dst).reshape(NC, N, HF)
    hs3 = _mid_call(agg2, dinv, b2.reshape(1, F), W3).reshape(2 * N, HF)
    agg3 = _scatter_kernel(hs3, src, dst).reshape(NC, N, HF)
    h, out = _fin_call(agg3, dinv, b3.reshape(1, F), fcW, fcb.reshape(1, NCLS))
    return (out, h)


# speedup vs baseline: 17.3236x; 1.0092x over previous
import functools

import jax
import jax.numpy as jnp
from jax import lax
from jax.experimental import pallas as pl
from jax.experimental.pallas import tpu as pltpu
from jax.experimental.pallas import tpu_sc as plsc

N = 10000
E = 160000
F = 256
HF = 128
NCLS = 64
NC = 2
NS = 16
ROWS_PER_TILE = N // NS
K = 40
EDGES_PER_TILE = E // NS
DEG_EDGES_PER_TILE = E // (NC * NS)
KD = 40
NCH = EDGES_PER_TILE // K
NCHD = DEG_EDGES_PER_TILE // KD
NBUF = 5
GROUPS = NCH // NBUF
RB = 2000

_mesh = plsc.VectorSubcoreMesh(core_axis_name="c", subcore_axis_name="s")
_sc_params = pltpu.CompilerParams(use_tc_tiling_on_sc=False)


@functools.partial(
    pl.kernel,
    out_type=jax.ShapeDtypeStruct((NC, N, 16), jnp.float32),
    mesh=_mesh,
    compiler_params=_sc_params,
    scratch_types=[
        pltpu.VMEM_SHARED((N, 16), jnp.float32),
        pltpu.VMEM((25, 16), jnp.float32),
        pltpu.VMEM((KD, 16), jnp.float32),
        pltpu.VMEM((NCHD, KD), jnp.int32),
        pltpu.SemaphoreType.DMA,
    ],
)
def _deg_kernel(dst_hbm, out_hbm, cnt_sh, zbuf, ones_v, didx2, dsem):
    c = lax.axis_index("c")
    s = lax.axis_index("s")
    w = c * NS + s

    def fill_zero(i, carry):
        zbuf[i, :] = jnp.zeros((16,), jnp.float32)
        return carry

    lax.fori_loop(0, 25, fill_zero, 0)

    def fill_one(i, carry):
        ones_v[i, :] = jnp.ones((16,), jnp.float32)
        return carry

    lax.fori_loop(0, KD, fill_one, 0)

    row0 = s * ROWS_PER_TILE

    def zero_out(i, carry):
        pltpu.sync_copy(zbuf, cnt_sh.at[pl.ds(row0 + i * 25, 25)])
        return carry

    lax.fori_loop(0, ROWS_PER_TILE // 25, zero_out, 0)
    plsc.subcore_barrier()

    base = w * DEG_EDGES_PER_TILE

    def ldx(i, carry):
        pltpu.async_copy(dst_hbm.at[pl.ds(base + i * KD, KD)],
                         didx2.at[i], dsem)
        return carry

    lax.fori_loop(0, NCHD, ldx, 0)

    def ldx_drain(i, carry):
        pltpu.make_async_copy(dst_hbm.at[pl.ds(base, KD)],
                              didx2.at[0], dsem).wait()
        return carry

    lax.fori_loop(0, NCHD, ldx_drain, 0)

    def fire(i, carry):
        pltpu.async_copy(ones_v, cnt_sh.at[didx2.at[i]], dsem, add=True)
        return carry

    lax.fori_loop(0, NCHD, fire, 0)

    def drain(i, carry):
        pltpu.make_async_copy(ones_v, cnt_sh.at[didx2.at[0]], dsem).wait()
        return carry

    lax.fori_loop(0, NCHD, drain, 0)
    plsc.subcore_barrier()

    pltpu.sync_copy(cnt_sh.at[pl.ds(row0, ROWS_PER_TILE)],
                    out_hbm.at[c, pl.ds(row0, ROWS_PER_TILE)])


@functools.partial(
    pl.kernel,
    out_type=jax.ShapeDtypeStruct((2 * N, HF), jnp.float32),
    mesh=_mesh,
    compiler_params=_sc_params,
    scratch_types=[
        pltpu.VMEM_SHARED((N, HF), jnp.float32),
        *([pltpu.VMEM((2, K), jnp.int32)] * (2 * NBUF)),
        *([pltpu.VMEM((K,), jnp.int32)] * (2 * NBUF)),
        *([pltpu.VMEM((K, HF), jnp.float32)] * NBUF),
        *([pltpu.SemaphoreType.DMA] * (4 * NBUF)),
    ],
)
def _scatter_kernel(hs_hbm, ei_hbm, out_hbm, acc_sh, *ring):
    IB = 2 * NBUF
    sds = ring[:IB]
    gis = ring[IB:2 * IB]
    gbufs = ring[2 * IB:2 * IB + NBUF]
    isems = ring[2 * IB + NBUF:3 * IB + NBUF]
    gsems = ring[3 * IB + NBUF:3 * IB + 2 * NBUF]
    ssems = ring[3 * IB + 2 * NBUF:]
    c = lax.axis_index("c")
    s = lax.axis_index("s")
    row0 = s * ROWS_PER_TILE
    half = c * N

    pltpu.sync_copy(hs_hbm.at[pl.ds(half + row0, ROWS_PER_TILE)],
                    acc_sh.at[pl.ds(row0, ROWS_PER_TILE)])
    ebase = s * EDGES_PER_TILE
    windows = sorted({min(j * 16, K - 16) for j in range((K + 15) // 16)})

    def ld_idx(chunk, q):
        nb = ebase + chunk * K
        pltpu.async_copy(ei_hbm.at[:, pl.ds(nb, K)], sds[q], isems[q])

    def wait_idx(q):
        pltpu.make_async_copy(ei_hbm.at[:, pl.ds(ebase, K)],
                              sds[q], isems[q]).wait()

    def wait_scatter(b, q):
        pltpu.make_async_copy(gbufs[b], acc_sh.at[sds[q].at[1]],
                              ssems[b]).wait()

    for q in range(IB):
        ld_idx(q, q)
    plsc.subcore_barrier()

    def outer(t, carry):
        for p in (0, 1):
            gd = []
            for b in range(NBUF):
                q = p * NBUF + b
                wait_idx(q)
                for w0 in windows:
                    gis[q][pl.ds(w0, 16)] = sds[q][0, pl.ds(w0, 16)] + half
                pq = (1 - p) * NBUF + b
                if p == 1:
                    wait_scatter(b, pq)
                else:
                    @pl.when(t > 0)
                    def _():
                        wait_scatter(b, pq)
                gd.append(pltpu.async_copy(hs_hbm.at[gis[q]], gbufs[b],
                                           gsems[b]))
            for b in range(NBUF):
                q = p * NBUF + b
                gd[b].wait()
                pltpu.async_copy(gbufs[b], acc_sh.at[sds[q].at[1]], ssems[b],
                                 add=True)
                pq = (1 - p) * NBUF + b
                nxt = (2 * t + p + 1) * NBUF + b
                if p == 0:
                    @pl.when(jnp.logical_and(t > 0, nxt < NCH))
                    def _():
                        ld_idx(nxt, pq)
                else:
                    @pl.when(nxt < NCH)
                    def _():
                        ld_idx(nxt, pq)
        return carry

    lax.fori_loop(0, NCH // (2 * NBUF), outer, 0)
    for b in range(NBUF):
        wait_scatter(b, NBUF + b)
    plsc.subcore_barrier()

    pltpu.sync_copy(acc_sh.at[pl.ds(row0, ROWS_PER_TILE)],
                    out_hbm.at[pl.ds(half + row0, ROWS_PER_TILE)])


def _dinv_of(dw_ref):
    return lax.rsqrt(dw_ref[0][:, 0:1] + dw_ref[1][:, 0:1] + 1.0)


_dw_spec = pl.BlockSpec((NC, RB, 16), lambda i: (0, i, 0))


def _mm1_body(x_ref, w_ref, dw_ref, o_ref):
    y = jnp.dot(x_ref[...], w_ref[...], preferred_element_type=jnp.float32)
    y = y * _dinv_of(dw_ref)
    o_ref[0] = y[:, :HF]
    o_ref[1] = y[:, HF:]


_mm1_call = pl.pallas_call(
    _mm1_body,
    grid=(N // RB,),
    in_specs=[
        pl.BlockSpec((RB, F), lambda i: (i, 0)),
        pl.BlockSpec((F, F), lambda i: (0, 0)),
        _dw_spec,
    ],
    out_specs=pl.BlockSpec((NC, RB, HF), lambda i: (0, i, 0)),
    out_shape=jax.ShapeDtypeStruct((NC, N, HF), jnp.float32),
)


def _mid_body(agg_ref, dw_ref, b_ref, w_ref, o_ref):
    dinv = _dinv_of(dw_ref)
    h = jnp.concatenate([agg_ref[0], agg_ref[1]], axis=1) * dinv + b_ref[...]
    y = jnp.dot(h, w_ref[...], preferred_element_type=jnp.float32) * dinv
    o_ref[0] = y[:, :HF]
    o_ref[1] = y[:, HF:]


_mid_call = pl.pallas_call(
    _mid_body,
    grid=(N // RB,),
    in_specs=[
        pl.BlockSpec((NC, RB, HF), lambda i: (0, i, 0)),
        _dw_spec,
        pl.BlockSpec((1, F), lambda i: (0, 0)),
        pl.BlockSpec((F, F), lambda i: (0, 0)),
    ],
    out_specs=pl.BlockSpec((NC, RB, HF), lambda i: (0, i, 0)),
    out_shape=jax.ShapeDtypeStruct((NC, N, HF), jnp.float32),
)


def _fin_body(agg_ref, dw_ref, b_ref, fcw_ref, fcb_ref, h_ref, o_ref):
    h = (jnp.concatenate([agg_ref[0], agg_ref[1]], axis=1) * _dinv_of(dw_ref)
         + b_ref[...])
    h_ref[...] = h
    o_ref[...] = lax.dot_general(
        h, fcw_ref[...], (((1,), (1,)), ((), ())),
        preferred_element_type=jnp.float32) + fcb_ref[...]


_fin_call = pl.pallas_call(
    _fin_body,
    grid=(N // RB,),
    in_specs=[
        pl.BlockSpec((NC, RB, HF), lambda i: (0, i, 0)),
        _dw_spec,
        pl.BlockSpec((1, F), lambda i: (0, 0)),
        pl.BlockSpec((NCLS, F), lambda i: (0, 0)),
        pl.BlockSpec((1, NCLS), lambda i: (0, 0)),
    ],
    out_specs=[
        pl.BlockSpec((RB, F), lambda i: (i, 0)),
        pl.BlockSpec((RB, NCLS), lambda i: (i, 0)),
    ],
    out_shape=[
        jax.ShapeDtypeStruct((N, F), jnp.float32),
        jax.ShapeDtypeStruct((N, NCLS), jnp.float32),
    ],
)


def kernel(x, edge_index, W1, b1, W2, b2, W3, b3, fcW, fcb):
    dst = edge_index[1]
    dw = _deg_kernel(dst)
    hs = _mm1_call(x, W1, dw).reshape(2 * N, HF)
    agg1 = _scatter_kernel(hs, edge_index).reshape(NC, N, HF)
    hs2 = _mid_call(agg1, dw, b1.reshape(1, F), W2).reshape(2 * N, HF)
    agg2 = _scatter_kernel(hs2, edge_index).reshape(NC, N, HF)
    hs3 = _mid_call(agg2, dw, b2.reshape(1, F), W3).reshape(2 * N, HF)
    agg3 = _scatter_kernel(hs3, edge_index).reshape(NC, N, HF)
    h, out = _fin_call(agg3, dw, b3.reshape(1, F), fcW, fcb.reshape(1, NCLS))
    return (out, h)


# speedup vs baseline: 17.3400x; 1.0009x over previous
import functools

import jax
import jax.numpy as jnp
from jax import lax
from jax.experimental import pallas as pl
from jax.experimental.pallas import tpu as pltpu
from jax.experimental.pallas import tpu_sc as plsc

N = 10000
E = 160000
F = 256
HF = 128
NCLS = 64
NC = 2
NS = 16
ROWS_PER_TILE = N // NS
K = 40
EDGES_PER_TILE = E // NS
DEG_EDGES_PER_TILE = E // (NC * NS)
KD = 40
NCH = EDGES_PER_TILE // K
NCHD = DEG_EDGES_PER_TILE // KD
NBUF = 5
GROUPS = NCH // NBUF
RB = 2000

_mesh = plsc.VectorSubcoreMesh(core_axis_name="c", subcore_axis_name="s")
_sc_params = pltpu.CompilerParams(use_tc_tiling_on_sc=False)


@functools.partial(
    pl.kernel,
    out_type=jax.ShapeDtypeStruct((NC, N, 16), jnp.float32),
    mesh=_mesh,
    compiler_params=_sc_params,
    scratch_types=[
        pltpu.VMEM_SHARED((N, 16), jnp.float32),
        pltpu.VMEM((25, 16), jnp.float32),
        pltpu.VMEM((KD, 16), jnp.float32),
        pltpu.VMEM((NCHD, KD), jnp.int32),
        pltpu.SemaphoreType.DMA,
    ],
)
def _deg_kernel(dst_hbm, out_hbm, cnt_sh, zbuf, ones_v, didx2, dsem):
    c = lax.axis_index("c")
    s = lax.axis_index("s")
    w = c * NS + s

    def fill_zero(i, carry):
        zbuf[i, :] = jnp.zeros((16,), jnp.float32)
        return carry

    lax.fori_loop(0, 25, fill_zero, 0)

    def fill_one(i, carry):
        ones_v[i, :] = jnp.ones((16,), jnp.float32)
        return carry

    lax.fori_loop(0, KD, fill_one, 0)

    row0 = s * ROWS_PER_TILE

    def zero_out(i, carry):
        pltpu.sync_copy(zbuf, cnt_sh.at[pl.ds(row0 + i * 25, 25)])
        return carry

    lax.fori_loop(0, ROWS_PER_TILE // 25, zero_out, 0)
    plsc.subcore_barrier()

    base = w * DEG_EDGES_PER_TILE

    def ldx(i, carry):
        pltpu.async_copy(dst_hbm.at[pl.ds(base + i * KD, KD)],
                         didx2.at[i], dsem)
        return carry

    lax.fori_loop(0, NCHD, ldx, 0)

    def ldx_drain(i, carry):
        pltpu.make_async_copy(dst_hbm.at[pl.ds(base, KD)],
                              didx2.at[0], dsem).wait()
        return carry

    lax.fori_loop(0, NCHD, ldx_drain, 0)

    def fire(i, carry):
        pltpu.async_copy(ones_v, cnt_sh.at[didx2.at[i]], dsem, add=True)
        return carry

    lax.fori_loop(0, NCHD, fire, 0)

    def drain(i, carry):
        pltpu.make_async_copy(ones_v, cnt_sh.at[didx2.at[0]], dsem).wait()
        return carry

    lax.fori_loop(0, NCHD, drain, 0)
    plsc.subcore_barrier()

    pltpu.sync_copy(cnt_sh.at[pl.ds(row0, ROWS_PER_TILE)],
                    out_hbm.at[c, pl.ds(row0, ROWS_PER_TILE)])


@functools.partial(
    pl.kernel,
    out_type=jax.ShapeDtypeStruct((2 * N, HF), jnp.float32),
    mesh=_mesh,
    compiler_params=_sc_params,
    scratch_types=[
        pltpu.VMEM_SHARED((N, HF), jnp.float32),
        *([pltpu.VMEM((2, K), jnp.int32)] * (2 * NBUF)),
        *([pltpu.VMEM((K,), jnp.int32)] * (2 * NBUF)),
        *([pltpu.VMEM((K, HF), jnp.float32)] * NBUF),
        *([pltpu.SemaphoreType.DMA] * (4 * NBUF)),
    ],
)
def _scatter_kernel(hs_hbm, ei_hbm, out_hbm, acc_sh, *ring):
    IB = 2 * NBUF
    sds = ring[:IB]
    gis = ring[IB:2 * IB]
    gbufs = ring[2 * IB:2 * IB + NBUF]
    isems = ring[2 * IB + NBUF:3 * IB + NBUF]
    gsems = ring[3 * IB + NBUF:3 * IB + 2 * NBUF]
    ssems = ring[3 * IB + 2 * NBUF:]
    c = lax.axis_index("c")
    s = lax.axis_index("s")
    row0 = s * ROWS_PER_TILE
    half = c * N

    init_cp = pltpu.async_copy(hs_hbm.at[pl.ds(half + row0, ROWS_PER_TILE)],
                               acc_sh.at[pl.ds(row0, ROWS_PER_TILE)],
                               isems[0])
    ebase = s * EDGES_PER_TILE
    windows = sorted({min(j * 16, K - 16) for j in range((K + 15) // 16)})

    def ld_idx(chunk, q):
        nb = ebase + chunk * K
        pltpu.async_copy(ei_hbm.at[:, pl.ds(nb, K)], sds[q], isems[q])

    def wait_idx(q):
        pltpu.make_async_copy(ei_hbm.at[:, pl.ds(ebase, K)],
                              sds[q], isems[q]).wait()

    def wait_scatter(b, q):
        pltpu.make_async_copy(gbufs[b], acc_sh.at[sds[q].at[1]],
                              ssems[b]).wait()

    for q in range(1, IB):
        ld_idx(q, q)
    init_cp.wait()
    ld_idx(0, 0)
    plsc.subcore_barrier()

    def outer(t, carry):
        for p in (0, 1):
            gd = []
            for b in range(NBUF):
                q = p * NBUF + b
                wait_idx(q)
                for w0 in windows:
                    gis[q][pl.ds(w0, 16)] = sds[q][0, pl.ds(w0, 16)] + half
                pq = (1 - p) * NBUF + b
                if p == 1:
                    wait_scatter(b, pq)
                else:
                    @pl.when(t > 0)
                    def _():
                        wait_scatter(b, pq)
                gd.append(pltpu.async_copy(hs_hbm.at[gis[q]], gbufs[b],
                                           gsems[b]))
            for b in range(NBUF):
                q = p * NBUF + b
                gd[b].wait()
                pltpu.async_copy(gbufs[b], acc_sh.at[sds[q].at[1]], ssems[b],
                                 add=True)
                pq = (1 - p) * NBUF + b
                nxt = (2 * t + p + 1) * NBUF + b
                if p == 0:
                    @pl.when(jnp.logical_and(t > 0, nxt < NCH))
                    def _():
                        ld_idx(nxt, pq)
                else:
                    @pl.when(nxt < NCH)
                    def _():
                        ld_idx(nxt, pq)
        return carry

    lax.fori_loop(0, NCH // (2 * NBUF), outer, 0)
    for b in range(NBUF):
        wait_scatter(b, NBUF + b)
    plsc.subcore_barrier()

    pltpu.sync_copy(acc_sh.at[pl.ds(row0, ROWS_PER_TILE)],
                    out_hbm.at[pl.ds(half + row0, ROWS_PER_TILE)])


def _dinv_of(dw_ref):
    return lax.rsqrt(dw_ref[0][:, 0:1] + dw_ref[1][:, 0:1] + 1.0)


_dw_spec = pl.BlockSpec((NC, RB, 16), lambda i: (0, i, 0))


def _mm1_body(x_ref, w_ref, dw_ref, o_ref):
    y = jnp.dot(x_ref[...], w_ref[...], preferred_element_type=jnp.float32)
    y = y * _dinv_of(dw_ref)
    o_ref[0] = y[:, :HF]
    o_ref[1] = y[:, HF:]


_mm1_call = pl.pallas_call(
    _mm1_body,
    grid=(N // RB,),
    in_specs=[
        pl.BlockSpec((RB, F), lambda i: (i, 0)),
        pl.BlockSpec((F, F), lambda i: (0, 0)),
        _dw_spec,
    ],
    out_specs=pl.BlockSpec((NC, RB, HF), lambda i: (0, i, 0)),
    out_shape=jax.ShapeDtypeStruct((NC, N, HF), jnp.float32),
)


def _mid_body(agg_ref, dw_ref, b_ref, w_ref, o_ref):
    dinv = _dinv_of(dw_ref)
    h = jnp.concatenate([agg_ref[0], agg_ref[1]], axis=1) * dinv + b_ref[...]
    y = jnp.dot(h, w_ref[...], preferred_element_type=jnp.float32) * dinv
    o_ref[0] = y[:, :HF]
    o_ref[1] = y[:, HF:]


_mid_call = pl.pallas_call(
    _mid_body,
    grid=(N // RB,),
    in_specs=[
        pl.BlockSpec((NC, RB, HF), lambda i: (0, i, 0)),
        _dw_spec,
        pl.BlockSpec((1, F), lambda i: (0, 0)),
        pl.BlockSpec((F, F), lambda i: (0, 0)),
    ],
    out_specs=pl.BlockSpec((NC, RB, HF), lambda i: (0, i, 0)),
    out_shape=jax.ShapeDtypeStruct((NC, N, HF), jnp.float32),
)


def _fin_body(agg_ref, dw_ref, b_ref, fcw_ref, fcb_ref, h_ref, o_ref):
    h = (jnp.concatenate([agg_ref[0], agg_ref[1]], axis=1) * _dinv_of(dw_ref)
         + b_ref[...])
    h_ref[...] = h
    o_ref[...] = lax.dot_general(
        h, fcw_ref[...], (((1,), (1,)), ((), ())),
        preferred_element_type=jnp.float32) + fcb_ref[...]


_fin_call = pl.pallas_call(
    _fin_body,
    grid=(N // RB,),
    in_specs=[
        pl.BlockSpec((NC, RB, HF), lambda i: (0, i, 0)),
        _dw_spec,
        pl.BlockSpec((1, F), lambda i: (0, 0)),
        pl.BlockSpec((NCLS, F), lambda i: (0, 0)),
        pl.BlockSpec((1, NCLS), lambda i: (0, 0)),
    ],
    out_specs=[
        pl.BlockSpec((RB, F), lambda i: (i, 0)),
        pl.BlockSpec((RB, NCLS), lambda i: (i, 0)),
    ],
    out_shape=[
        jax.ShapeDtypeStruct((N, F), jnp.float32),
        jax.ShapeDtypeStruct((N, NCLS), jnp.float32),
    ],
)


def kernel(x, edge_index, W1, b1, W2, b2, W3, b3, fcW, fcb):
    dst = edge_index[1]
    dw = _deg_kernel(dst)
    hs = _mm1_call(x, W1, dw).reshape(2 * N, HF)
    agg1 = _scatter_kernel(hs, edge_index).reshape(NC, N, HF)
    hs2 = _mid_call(agg1, dw, b1.reshape(1, F), W2).reshape(2 * N, HF)
    agg2 = _scatter_kernel(hs2, edge_index).reshape(NC, N, HF)
    hs3 = _mid_call(agg2, dw, b2.reshape(1, F), W3).reshape(2 * N, HF)
    agg3 = _scatter_kernel(hs3, edge_index).reshape(NC, N, HF)
    h, out = _fin_call(agg3, dw, b3.reshape(1, F), fcW, fcb.reshape(1, NCLS))
    return (out, h)
